# Initial kernel scaffold; baseline (speedup 1.0000x reference)
import functools

import jax
import jax.numpy as jnp
from jax import lax
from jax.experimental import pallas as pl
from jax.experimental.pallas import tpu as pltpu
from jax.experimental.pallas import tpu_sc as plsc

NC = 2
NS = 16
NW = NC * NS
C = 128
G = 64
PG = 72


def _mesh():
    return plsc.VectorSubcoreMesh(
        core_axis_name="c", subcore_axis_name="s", num_cores=NC, num_subcores=NS
    )


def _zero_vmem(ref, rows):
    k = ref.shape[1] // 16

    def body(i, _):
        for t in range(k):
            ref[i, pl.ds(16 * t, 16)] = jnp.zeros((16,), jnp.float32)
        return 0

    lax.fori_loop(0, rows, body, 0)


def _sc_degree(dsts2d, R, EC):
    rpt = R // NS

    @functools.partial(
        pl.kernel,
        out_type=jax.ShapeDtypeStruct((NC, R, 16), jnp.float32),
        mesh=_mesh(),
        scratch_types=[
            pltpu.VMEM((EC, C), jnp.int32),
            pltpu.VMEM((C, 16), jnp.float32),
            pltpu.VMEM((rpt, 16), jnp.float32),
            pltpu.VMEM_SHARED((R, 16), jnp.float32),
        ],
    )
    def k(dsts_hbm, out_hbm, didx, ones_v, zbuf, deg_sh):
        cid = lax.axis_index("c")
        sid = lax.axis_index("s")
        w = sid * NC + cid

        def fill_ones(i, _):
            ones_v[i] = jnp.full((16,), 1.0, jnp.float32)
            return 0

        lax.fori_loop(0, C, fill_ones, 0)
        _zero_vmem(zbuf, rpt)
        pltpu.sync_copy(zbuf, deg_sh.at[pl.ds(sid * rpt, rpt)])
        plsc.subcore_barrier()

        pltpu.sync_copy(dsts_hbm.at[pl.ds(w * EC, EC)], didx)

        def body(j, _):
            pltpu.sync_copy(ones_v, deg_sh.at[didx.at[j]], add=True)
            return 0

        lax.fori_loop(0, EC, body, 0)
        plsc.subcore_barrier()
        pltpu.sync_copy(
            deg_sh.at[pl.ds(sid * rpt, rpt)],
            out_hbm.at[cid, pl.ds(sid * rpt, rpt)],
        )

    return k(dsts2d)


def _sc_edge_scatter(hs, srcs2d, dsts2d, R, EC):
    rpt = R // NS

    @functools.partial(
        pl.kernel,
        out_type=jax.ShapeDtypeStruct((NC, R, 128), jnp.float32),
        mesh=_mesh(),
        scratch_types=[
            pltpu.VMEM((EC, C), jnp.int32),
            pltpu.VMEM((EC, C), jnp.int32),
            pltpu.VMEM((C, 128), jnp.float32),
            pltpu.VMEM_SHARED((R, 128), jnp.float32),
        ],
    )
    def k(hs_hbm, srcs_hbm, dsts_hbm, out_hbm, sidx, didx, rows, acc_sh):
        cid = lax.axis_index("c")
        sid = lax.axis_index("s")
        w = sid * NC + cid

        _zero_vmem(rows, C)
        for t in range(rpt // C):
            pltpu.sync_copy(rows, acc_sh.at[pl.ds(sid * rpt + t * C, C)])
        plsc.subcore_barrier()

        pltpu.sync_copy(srcs_hbm.at[pl.ds(w * EC, EC)], sidx)
        pltpu.sync_copy(dsts_hbm.at[pl.ds(w * EC, EC)], didx)

        def body(j, _):
            pltpu.sync_copy(hs_hbm.at[sidx.at[j]], rows)
            pltpu.sync_copy(rows, acc_sh.at[didx.at[j]], add=True)
            return 0

        lax.fori_loop(0, EC, body, 0)
        plsc.subcore_barrier()
        pltpu.sync_copy(
            acc_sh.at[pl.ds(sid * rpt, rpt)],
            out_hbm.at[cid, pl.ds(sid * rpt, rpt)],
        )

    return k(hs, srcs2d, dsts2d)


def _sc_pool(h2, batch2d, R):
    n_chunks = R // C
    base_per_w = n_chunks // NW
    extra = n_chunks - base_per_w * NW

    @functools.partial(
        pl.kernel,
        out_type=(
            jax.ShapeDtypeStruct((NC, PG, 128), jnp.float32),
            jax.ShapeDtypeStruct((NC, PG, 16), jnp.float32),
        ),
        mesh=_mesh(),
        scratch_types=[
            pltpu.VMEM((C,), jnp.int32),
            pltpu.VMEM((C, 128), jnp.float32),
            pltpu.VMEM((C, 16), jnp.float32),
            pltpu.VMEM((PG, 128), jnp.float32),
            pltpu.VMEM_SHARED((PG, 128), jnp.float32),
            pltpu.VMEM_SHARED((PG, 16), jnp.float32),
        ],
    )
    def k(h2_hbm, b_hbm, sums_hbm, cnts_hbm, bidx, rows, ones_v, zbuf,
          sums_sh, cnts_sh):
        cid = lax.axis_index("c")
        sid = lax.axis_index("s")
        w = sid * NC + cid

        def fill_ones(i, _):
            ones_v[i] = jnp.full((16,), 1.0, jnp.float32)
            return 0

        lax.fori_loop(0, C, fill_ones, 0)
        _zero_vmem(zbuf, PG)

        @pl.when(sid == 0)
        def _():
            pltpu.sync_copy(zbuf, sums_sh)

        @pl.when(sid == 1)
        def _():
            pltpu.sync_copy(zbuf.at[:, pl.ds(0, 16)], cnts_sh)

        plsc.subcore_barrier()

        nw = jnp.where(w < extra, base_per_w + 1, base_per_w)
        start = w * base_per_w + jnp.minimum(w, extra)

        def body(t, _):
            c = start + t
            pltpu.sync_copy(b_hbm.at[c], bidx)
            pltpu.sync_copy(h2_hbm.at[pl.ds(c * C, C)], rows)
            pltpu.sync_copy(rows, sums_sh.at[bidx], add=True)
            pltpu.sync_copy(ones_v, cnts_sh.at[bidx], add=True)
            return 0

        lax.fori_loop(0, nw, body, 0)
        plsc.subcore_barrier()

        @pl.when(sid == 0)
        def _():
            pltpu.sync_copy(sums_sh, sums_hbm.at[cid])

        @pl.when(sid == 1)
        def _():
            pltpu.sync_copy(cnts_sh, cnts_hbm.at[cid])

    return k(h2, batch2d)


def _dinv_from(deg_ref):
    deg = deg_ref[0, :, 0:1] + deg_ref[1, :, 0:1] + 1.0
    return lax.rsqrt(deg)


def _tc1(degp, x_pad, W1, R, BLK):
    def body(deg_ref, x_ref, w_ref, o_ref):
        dinv = _dinv_from(deg_ref)
        h = jnp.dot(x_ref[...], w_ref[...], preferred_element_type=jnp.float32)
        o_ref[...] = h * dinv

    return pl.pallas_call(
        body,
        grid=(R // BLK,),
        in_specs=[
            pl.BlockSpec((2, BLK, 16), lambda i: (0, i, 0)),
            pl.BlockSpec((BLK, 128), lambda i: (i, 0)),
            pl.BlockSpec((128, 128), lambda i: (0, 0)),
        ],
        out_specs=pl.BlockSpec((BLK, 128), lambda i: (i, 0)),
        out_shape=jax.ShapeDtypeStruct((R, 128), jnp.float32),
    )(degp, x_pad, W1)


def _tc2(degp, accp, hs1, W2, b1, R, BLK):
    def body(deg_ref, acc_ref, hs_ref, w_ref, b_ref, o_ref):
        dinv = _dinv_from(deg_ref)
        t = (acc_ref[0] + acc_ref[1] + hs_ref[...]) * dinv + b_ref[...]
        h1 = jnp.maximum(t, 0.0)
        o_ref[...] = (
            jnp.dot(h1, w_ref[...], preferred_element_type=jnp.float32) * dinv
        )

    return pl.pallas_call(
        body,
        grid=(R // BLK,),
        in_specs=[
            pl.BlockSpec((2, BLK, 16), lambda i: (0, i, 0)),
            pl.BlockSpec((2, BLK, 128), lambda i: (0, i, 0)),
            pl.BlockSpec((BLK, 128), lambda i: (i, 0)),
            pl.BlockSpec((128, 128), lambda i: (0, 0)),
            pl.BlockSpec((1, 128), lambda i: (0, 0)),
        ],
        out_specs=pl.BlockSpec((BLK, 128), lambda i: (i, 0)),
        out_shape=jax.ShapeDtypeStruct((R, 128), jnp.float32),
    )(degp, accp, hs1, W2, b1)


def _tc3(degp, accp, hs2, b2, R, BLK):
    def body(deg_ref, acc_ref, hs_ref, b_ref, o_ref):
        dinv = _dinv_from(deg_ref)
        o_ref[...] = (
            (acc_ref[0] + acc_ref[1] + hs_ref[...]) * dinv + b_ref[...]
        )

    return pl.pallas_call(
        body,
        grid=(R // BLK,),
        in_specs=[
            pl.BlockSpec((2, BLK, 16), lambda i: (0, i, 0)),
            pl.BlockSpec((2, BLK, 128), lambda i: (0, i, 0)),
            pl.BlockSpec((BLK, 128), lambda i: (i, 0)),
            pl.BlockSpec((1, 128), lambda i: (0, 0)),
        ],
        out_specs=pl.BlockSpec((BLK, 128), lambda i: (i, 0)),
        out_shape=jax.ShapeDtypeStruct((R, 128), jnp.float32),
    )(degp, accp, hs2, b2)


def _tc4(sums, cnts, Wc1, bc1, Wc2, bc2, out_dim):
    def body(s_ref, c_ref, w1_ref, b1_ref, w2_ref, b2_ref, o_ref):
        s = s_ref[0, :G, :] + s_ref[1, :G, :]
        c = c_ref[0, :G, 0:1] + c_ref[1, :G, 0:1]
        gf = s / jnp.maximum(c, 1.0)
        z = jnp.dot(gf, w1_ref[...], preferred_element_type=jnp.float32)
        z = jnp.maximum(z + b1_ref[...], 0.0)
        o_ref[...] = (
            jnp.dot(z, w2_ref[...], preferred_element_type=jnp.float32)
            + b2_ref[...]
        )

    return pl.pallas_call(
        body,
        out_shape=jax.ShapeDtypeStruct((G, out_dim), jnp.float32),
    )(sums, cnts, Wc1, bc1, Wc2, bc2)


def kernel(x, edge_index, batch, W1, b1, W2, b2, Wc1, bc1, Wc2, bc2):
    N, D = x.shape
    E = edge_index.shape[1]
    out_dim = Wc2.shape[1]

    R = ((N + 1) + NS * C - 1) // (NS * C) * (NS * C)
    EC = -(-E // (NW * C))
    E_pad = NW * C * EC

    pad_e = jnp.full((E_pad - E,), N, dtype=jnp.int32)
    srcs2d = jnp.concatenate([edge_index[0], pad_e]).reshape(NW * EC, C)
    dsts2d = jnp.concatenate([edge_index[1], pad_e]).reshape(NW * EC, C)
    x_pad = jnp.pad(x, ((0, R - N), (0, 0)))
    batch2d = jnp.concatenate(
        [batch.astype(jnp.int32), jnp.full((R - N,), G, dtype=jnp.int32)]
    ).reshape(R // C, C)

    BLK = 640
    degp = _sc_degree(dsts2d, R, EC)
    hs1 = _tc1(degp, x_pad, W1, R, BLK)
    accp1 = _sc_edge_scatter(hs1, srcs2d, dsts2d, R, EC)
    hs2 = _tc2(degp, accp1, hs1, W2, b1.reshape(1, -1), R, BLK)
    accp2 = _sc_edge_scatter(hs2, srcs2d, dsts2d, R, EC)
    h2 = _tc3(degp, accp2, hs2, b2.reshape(1, -1), R, BLK)
    sums, cnts = _sc_pool(h2, batch2d, R)
    logits = _tc4(
        sums, cnts, Wc1, bc1.reshape(1, -1), Wc2, bc2.reshape(1, -1), out_dim
    )
    return logits


# speedup vs baseline: 7.3912x; 7.3912x over previous
import functools

import jax
import jax.numpy as jnp
from jax import lax
from jax.experimental import pallas as pl
from jax.experimental.pallas import tpu as pltpu
from jax.experimental.pallas import tpu_sc as plsc

NC = 2
NS = 16
NW = NC * NS
C = 128
G = 64
PG = 72


def _mesh():
    return plsc.VectorSubcoreMesh(
        core_axis_name="c", subcore_axis_name="s", num_cores=NC, num_subcores=NS
    )


def _zero_vmem(ref, rows):
    k = ref.shape[1] // 16

    def body(i, _):
        for t in range(k):
            ref[i, pl.ds(16 * t, 16)] = jnp.zeros((16,), jnp.float32)
        return 0

    lax.fori_loop(0, rows, body, 0)


def _sc_degree(dsts2d, R, EC):
    rpt = R // NS

    CH = 8

    @functools.partial(
        pl.kernel,
        out_type=jax.ShapeDtypeStruct((NC * R, 128), jnp.float32),
        mesh=_mesh(),
        scratch_types=[
            pltpu.VMEM((CH, C), jnp.int32),
            pltpu.VMEM((C, 128), jnp.float32),
            pltpu.VMEM_SHARED((R, 128), jnp.float32),
        ],
    )
    def k(dsts_hbm, out_hbm, didx, buf, deg_sh):
        cid = lax.axis_index("c")
        sid = lax.axis_index("s")
        w = sid * NC + cid

        _zero_vmem(buf, C)
        for t in range(rpt // C):
            pltpu.sync_copy(buf, deg_sh.at[pl.ds(sid * rpt + t * C, C)])
        plsc.subcore_barrier()

        def fill_ones(i, _):
            for t in range(8):
                buf[i, pl.ds(16 * t, 16)] = jnp.full((16,), 1.0, jnp.float32)
            return 0

        lax.fori_loop(0, C, fill_ones, 0)

        def outer(g, _):
            pltpu.sync_copy(dsts_hbm.at[pl.ds(w * EC + g * CH, CH)], didx)

            def inner(j, _):
                pltpu.sync_copy(buf, deg_sh.at[didx.at[j]], add=True)
                return 0

            lax.fori_loop(0, CH, inner, 0)
            return 0

        lax.fori_loop(0, EC // CH, outer, 0)
        plsc.subcore_barrier()
        for t in range(rpt // C):
            pltpu.sync_copy(deg_sh.at[pl.ds(sid * rpt + t * C, C)], buf)
            pltpu.sync_copy(
                buf, out_hbm.at[pl.ds(cid * R + sid * rpt + t * C, C)]
            )

    return k(dsts2d).reshape(NC, R, 128)


def _sc_edge_scatter(hs, srcs2d, dsts2d, R, EC):
    rpt = R // NS
    CH = 8

    @functools.partial(
        pl.kernel,
        out_type=jax.ShapeDtypeStruct((NC * R, 128), jnp.float32),
        mesh=_mesh(),
        scratch_types=[
            pltpu.VMEM((CH, C), jnp.int32),
            pltpu.VMEM((CH, C), jnp.int32),
            pltpu.VMEM((C, 128), jnp.float32),
            pltpu.VMEM_SHARED((R, 128), jnp.float32),
        ],
    )
    def k(hs_hbm, srcs_hbm, dsts_hbm, out_hbm, sidx, didx, rows, acc_sh):
        cid = lax.axis_index("c")
        sid = lax.axis_index("s")
        w = sid * NC + cid

        _zero_vmem(rows, C)
        for t in range(rpt // C):
            pltpu.sync_copy(rows, acc_sh.at[pl.ds(sid * rpt + t * C, C)])
        plsc.subcore_barrier()

        def outer(g, _):
            pltpu.sync_copy(srcs_hbm.at[pl.ds(w * EC + g * CH, CH)], sidx)
            pltpu.sync_copy(dsts_hbm.at[pl.ds(w * EC + g * CH, CH)], didx)

            def inner(j, _):
                pltpu.sync_copy(hs_hbm.at[sidx.at[j]], rows)
                pltpu.sync_copy(rows, acc_sh.at[didx.at[j]], add=True)
                return 0

            lax.fori_loop(0, CH, inner, 0)
            return 0

        lax.fori_loop(0, EC // CH, outer, 0)
        plsc.subcore_barrier()
        for t in range(rpt // C):
            pltpu.sync_copy(acc_sh.at[pl.ds(sid * rpt + t * C, C)], rows)
            pltpu.sync_copy(
                rows, out_hbm.at[pl.ds(cid * R + sid * rpt + t * C, C)]
            )

    return k(hs, srcs2d, dsts2d).reshape(NC, R, 128)


def _sc_pool(h2, batch3d, R):
    n_chunks = R // C
    base_per_w = n_chunks // NW
    extra = n_chunks - base_per_w * NW

    @functools.partial(
        pl.kernel,
        out_type=(
            jax.ShapeDtypeStruct((NC * PG, 128), jnp.float32),
            jax.ShapeDtypeStruct((NC * PG, 128), jnp.float32),
        ),
        mesh=_mesh(),
        scratch_types=[
            pltpu.VMEM((1, C), jnp.int32),
            pltpu.VMEM((C, 128), jnp.float32),
            pltpu.VMEM((C, 128), jnp.float32),
            pltpu.VMEM((PG, 128), jnp.float32),
            pltpu.VMEM_SHARED((PG, 128), jnp.float32),
            pltpu.VMEM_SHARED((PG, 128), jnp.float32),
        ],
    )
    def k(h2_hbm, b_hbm, sums_hbm, cnts_hbm, bidx, rows, ones_v, zbuf,
          sums_sh, cnts_sh):
        cid = lax.axis_index("c")
        sid = lax.axis_index("s")
        w = sid * NC + cid

        def fill_ones(i, _):
            for t in range(8):
                ones_v[i, pl.ds(16 * t, 16)] = jnp.full(
                    (16,), 1.0, jnp.float32)
            return 0

        lax.fori_loop(0, C, fill_ones, 0)
        _zero_vmem(zbuf, PG)

        @pl.when(sid == 0)
        def _():
            pltpu.sync_copy(zbuf, sums_sh)

        @pl.when(sid == 1)
        def _():
            pltpu.sync_copy(zbuf, cnts_sh)

        plsc.subcore_barrier()

        nw = jnp.where(w < extra, base_per_w + 1, base_per_w)
        start = w * base_per_w + jnp.minimum(w, extra)

        def body(t, _):
            c = start + t
            pltpu.sync_copy(b_hbm.at[c], bidx)
            pltpu.sync_copy(h2_hbm.at[pl.ds(c * C, C)], rows)
            pltpu.sync_copy(rows, sums_sh.at[bidx.at[0]], add=True)
            pltpu.sync_copy(ones_v, cnts_sh.at[bidx.at[0]], add=True)
            return 0

        lax.fori_loop(0, nw, body, 0)
        plsc.subcore_barrier()

        @pl.when(sid == 0)
        def _():
            pltpu.sync_copy(sums_sh, zbuf)
            pltpu.sync_copy(zbuf, sums_hbm.at[pl.ds(cid * PG, PG)])

        @pl.when(sid == 1)
        def _():
            pltpu.sync_copy(cnts_sh, zbuf)
            pltpu.sync_copy(zbuf, cnts_hbm.at[pl.ds(cid * PG, PG)])

    sums, cnts = k(h2, batch3d)
    return sums.reshape(NC, PG, 128), cnts.reshape(NC, PG, 128)


def _dinv_from(deg_ref):
    deg = deg_ref[0, :, 0:1] + deg_ref[1, :, 0:1] + 1.0
    return lax.rsqrt(deg)


def _tc1(degp, x_pad, W1, R, BLK):
    def body(deg_ref, x_ref, w_ref, o_ref):
        dinv = _dinv_from(deg_ref)
        h = jnp.dot(x_ref[...], w_ref[...], preferred_element_type=jnp.float32)
        o_ref[...] = h * dinv

    return pl.pallas_call(
        body,
        grid=(R // BLK,),
        in_specs=[
            pl.BlockSpec((2, BLK, 128), lambda i: (0, i, 0)),
            pl.BlockSpec((BLK, 128), lambda i: (i, 0)),
            pl.BlockSpec((128, 128), lambda i: (0, 0)),
        ],
        out_specs=pl.BlockSpec((BLK, 128), lambda i: (i, 0)),
        out_shape=jax.ShapeDtypeStruct((R, 128), jnp.float32),
    )(degp, x_pad, W1)


def _tc2(degp, accp, hs1, W2, b1, R, BLK):
    def body(deg_ref, acc_ref, hs_ref, w_ref, b_ref, o_ref):
        dinv = _dinv_from(deg_ref)
        t = (acc_ref[0] + acc_ref[1] + hs_ref[...]) * dinv + b_ref[...]
        h1 = jnp.maximum(t, 0.0)
        o_ref[...] = (
            jnp.dot(h1, w_ref[...], preferred_element_type=jnp.float32) * dinv
        )

    return pl.pallas_call(
        body,
        grid=(R // BLK,),
        in_specs=[
            pl.BlockSpec((2, BLK, 128), lambda i: (0, i, 0)),
            pl.BlockSpec((2, BLK, 128), lambda i: (0, i, 0)),
            pl.BlockSpec((BLK, 128), lambda i: (i, 0)),
            pl.BlockSpec((128, 128), lambda i: (0, 0)),
            pl.BlockSpec((1, 128), lambda i: (0, 0)),
        ],
        out_specs=pl.BlockSpec((BLK, 128), lambda i: (i, 0)),
        out_shape=jax.ShapeDtypeStruct((R, 128), jnp.float32),
    )(degp, accp, hs1, W2, b1)


def _tc3(degp, accp, hs2, b2, R, BLK):
    def body(deg_ref, acc_ref, hs_ref, b_ref, o_ref):
        dinv = _dinv_from(deg_ref)
        o_ref[...] = (
            (acc_ref[0] + acc_ref[1] + hs_ref[...]) * dinv + b_ref[...]
        )

    return pl.pallas_call(
        body,
        grid=(R // BLK,),
        in_specs=[
            pl.BlockSpec((2, BLK, 128), lambda i: (0, i, 0)),
            pl.BlockSpec((2, BLK, 128), lambda i: (0, i, 0)),
            pl.BlockSpec((BLK, 128), lambda i: (i, 0)),
            pl.BlockSpec((1, 128), lambda i: (0, 0)),
        ],
        out_specs=pl.BlockSpec((BLK, 128), lambda i: (i, 0)),
        out_shape=jax.ShapeDtypeStruct((R, 128), jnp.float32),
    )(degp, accp, hs2, b2)


def _tc4(sums, cnts, Wc1, bc1, Wc2, bc2, out_dim):
    def body(s_ref, c_ref, w1_ref, b1_ref, w2_ref, b2_ref, o_ref):
        s = s_ref[0, :G, :] + s_ref[1, :G, :]
        c = c_ref[0, :G, 0:1] + c_ref[1, :G, 0:1]
        gf = s / jnp.maximum(c, 1.0)
        z = jnp.dot(gf, w1_ref[...], preferred_element_type=jnp.float32)
        z = jnp.maximum(z + b1_ref[...], 0.0)
        o_ref[...] = (
            jnp.dot(z, w2_ref[...], preferred_element_type=jnp.float32)
            + b2_ref[...]
        )

    return pl.pallas_call(
        body,
        out_shape=jax.ShapeDtypeStruct((G, out_dim), jnp.float32),
    )(sums, cnts, Wc1, bc1, Wc2, bc2)


def kernel(x, edge_index, batch, W1, b1, W2, b2, Wc1, bc1, Wc2, bc2):
    N, D = x.shape
    E = edge_index.shape[1]
    out_dim = Wc2.shape[1]

    R = ((N + 1) + NS * C - 1) // (NS * C) * (NS * C)
    EC = (-(-E // (NW * C)) + 7) // 8 * 8
    E_pad = NW * C * EC

    pad_e = jnp.full((E_pad - E,), N, dtype=jnp.int32)
    srcs2d = jnp.concatenate([edge_index[0], pad_e]).reshape(NW * EC, C)
    dsts2d = jnp.concatenate([edge_index[1], pad_e]).reshape(NW * EC, C)
    x_pad = jnp.pad(x, ((0, R - N), (0, 0)))
    batch3d = jnp.concatenate(
        [batch.astype(jnp.int32), jnp.full((R - N,), G, dtype=jnp.int32)]
    ).reshape(R // C, 1, C)

    BLK = 640
    degp = _sc_degree(dsts2d, R, EC)
    hs1 = _tc1(degp, x_pad, W1, R, BLK)
    accp1 = _sc_edge_scatter(hs1, srcs2d, dsts2d, R, EC)
    hs2 = _tc2(degp, accp1, hs1, W2, b1.reshape(1, -1), R, BLK)
    accp2 = _sc_edge_scatter(hs2, srcs2d, dsts2d, R, EC)
    h2 = _tc3(degp, accp2, hs2, b2.reshape(1, -1), R, BLK)
    sums, cnts = _sc_pool(h2, batch3d, R)
    logits = _tc4(
        sums, cnts, Wc1, bc1.reshape(1, -1), Wc2, bc2.reshape(1, -1), out_dim
    )
    return logits


# speedup vs baseline: 8.1776x; 1.1064x over previous
import functools

import jax
import jax.numpy as jnp
from jax import lax
from jax.experimental import pallas as pl
from jax.experimental.pallas import tpu as pltpu
from jax.experimental.pallas import tpu_sc as plsc

NC = 2
NS = 16
NW = NC * NS
C = 128
G = 64
PG = 72


def _mesh():
    return plsc.VectorSubcoreMesh(
        core_axis_name="c", subcore_axis_name="s", num_cores=NC, num_subcores=NS
    )


def _zero_vmem(ref, rows):
    k = ref.shape[1] // 16

    def body(i, _):
        for t in range(k):
            ref[i, pl.ds(16 * t, 16)] = jnp.zeros((16,), jnp.float32)
        return 0

    lax.fori_loop(0, rows, body, 0)


def _sc_degree(dsts2d, R, EC):
    rpt = R // NS

    CH = 8

    @functools.partial(
        pl.kernel,
        out_type=jax.ShapeDtypeStruct((NC * R, 128), jnp.float32),
        mesh=_mesh(),
        scratch_types=[
            pltpu.VMEM((2, CH, C), jnp.int32),
            pltpu.VMEM((C, 128), jnp.float32),
            pltpu.VMEM_SHARED((R, 128), jnp.float32),
            pltpu.SemaphoreType.DMA,
            pltpu.SemaphoreType.DMA,
        ],
    )
    def k(dsts_hbm, out_hbm, didx, buf, deg_sh, ssem, isem):
        cid = lax.axis_index("c")
        sid = lax.axis_index("s")
        w = sid * NC + cid
        NG = EC // CH

        _zero_vmem(buf, C)
        for t in range(rpt // C):
            pltpu.sync_copy(buf, deg_sh.at[pl.ds(sid * rpt + t * C, C)])
        plsc.subcore_barrier()

        def fill_ones(i, _):
            for t in range(8):
                buf[i, pl.ds(16 * t, 16)] = jnp.full((16,), 1.0, jnp.float32)
            return 0

        lax.fori_loop(0, C, fill_ones, 0)
        pltpu.sync_copy(dsts_hbm.at[pl.ds(w * EC, CH)], didx.at[0])

        def outer(g, _):
            pg = lax.rem(g, 2)
            npg = 1 - pg

            def refill():
                return pltpu.make_async_copy(
                    dsts_hbm.at[pl.ds(w * EC + (g + 1) * CH, CH)],
                    didx.at[npg], isem)

            @pl.when(g + 1 < NG)
            def _():
                refill().start()

            def sca(jl):
                return pltpu.make_async_copy(
                    buf, deg_sh.at[didx.at[pg, jl]], ssem)

            for jl in range(CH):
                sca(jl).start(add=True)
            for jl in range(CH):
                sca(jl).wait()

            @pl.when(g + 1 < NG)
            def _():
                refill().wait()

            return 0

        lax.fori_loop(0, EC // CH, outer, 0)
        plsc.subcore_barrier()
        for t in range(rpt // C):
            pltpu.sync_copy(deg_sh.at[pl.ds(sid * rpt + t * C, C)], buf)
            pltpu.sync_copy(
                buf, out_hbm.at[pl.ds(cid * R + sid * rpt + t * C, C)]
            )

    return k(dsts2d).reshape(NC, R, 128)


def _sc_edge_scatter(hs, srcs2d, dsts2d, R, EC, EC0, EC1):
    rpt = R // NS
    CH = 8

    @functools.partial(
        pl.kernel,
        out_type=jax.ShapeDtypeStruct((NC * R, 128), jnp.float32),
        mesh=_mesh(),
        scratch_types=[
            pltpu.VMEM((2, CH, C), jnp.int32),
            pltpu.VMEM((2, CH, C), jnp.int32),
            pltpu.VMEM((2, C, 128), jnp.float32),
            pltpu.VMEM_SHARED((R, 128), jnp.float32),
            pltpu.SemaphoreType.DMA,
            pltpu.SemaphoreType.DMA,
            pltpu.SemaphoreType.DMA,
            pltpu.SemaphoreType.DMA,
            pltpu.SemaphoreType.DMA,
        ],
    )
    def k(hs_hbm, srcs_hbm, dsts_hbm, out_hbm, sidx, didx, rows, acc_sh,
          gsem0, gsem1, ssem0, ssem1, isem):
        cid = lax.axis_index("c")
        sid = lax.axis_index("s")
        gsems = (gsem0, gsem1)
        ssems = (ssem0, ssem1)
        ng = jnp.where(cid == 0, EC0 // CH, EC1 // CH)
        chunk0 = jnp.where(cid == 0, sid * EC0, NS * EC0 + sid * EC1)

        def zero_rows0(i, _):
            for t in range(8):
                rows[0, i, pl.ds(16 * t, 16)] = jnp.zeros((16,), jnp.float32)
            return 0

        lax.fori_loop(0, C, zero_rows0, 0)
        for t in range(rpt // C):
            pltpu.sync_copy(rows.at[0], acc_sh.at[pl.ds(sid * rpt + t * C, C)])
        plsc.subcore_barrier()

        pltpu.sync_copy(srcs_hbm.at[pl.ds(chunk0, CH)], sidx.at[0])
        pltpu.sync_copy(dsts_hbm.at[pl.ds(chunk0, CH)], didx.at[0])

        def group(g, _):
            pg = lax.rem(g, 2)
            npg = 1 - pg

            def refill(buf, hbm, off):
                return pltpu.make_async_copy(
                    hbm.at[pl.ds(off, CH)], buf.at[npg], isem)

            nxt = chunk0 + (g + 1) * CH

            @pl.when(g + 1 < ng)
            def _():
                refill(sidx, srcs_hbm, nxt).start()
                refill(didx, dsts_hbm, nxt).start()

            def gat(jl, b):
                return pltpu.make_async_copy(
                    hs_hbm.at[sidx.at[pg, jl]], rows.at[b], gsems[b])

            def sca(jl, b):
                return pltpu.make_async_copy(
                    rows.at[b], acc_sh.at[didx.at[pg, jl]], ssems[b])

            gat(0, 0).start()
            for jl in range(CH):
                b = jl % 2
                nb = (jl + 1) % 2
                if jl + 1 < CH:
                    if jl >= 1:
                        sca(jl - 1, nb).wait()
                    gat(jl + 1, nb).start()
                gat(jl, b).wait()
                sca(jl, b).start(add=True)
            sca(CH - 2, 0).wait()
            sca(CH - 1, 1).wait()

            @pl.when(g + 1 < ng)
            def _():
                refill(sidx, srcs_hbm, nxt).wait()
                refill(didx, dsts_hbm, nxt).wait()

            return 0

        lax.fori_loop(0, ng, group, 0)
        plsc.subcore_barrier()
        for t in range(rpt // C):
            pltpu.sync_copy(acc_sh.at[pl.ds(sid * rpt + t * C, C)], rows.at[0])
            pltpu.sync_copy(
                rows.at[0],
                out_hbm.at[pl.ds(cid * R + sid * rpt + t * C, C)],
            )

    return k(hs, srcs2d, dsts2d).reshape(NC, R, 128)


def _sc_pool(h2, batch3d, R):
    n_chunks = R // C
    base_per_w = n_chunks // NW
    extra = n_chunks - base_per_w * NW

    @functools.partial(
        pl.kernel,
        out_type=(
            jax.ShapeDtypeStruct((NC * PG, 128), jnp.float32),
            jax.ShapeDtypeStruct((NC * PG, 128), jnp.float32),
        ),
        mesh=_mesh(),
        scratch_types=[
            pltpu.VMEM((1, C), jnp.int32),
            pltpu.VMEM((C, 128), jnp.float32),
            pltpu.VMEM((C, 128), jnp.float32),
            pltpu.VMEM((PG, 128), jnp.float32),
            pltpu.VMEM_SHARED((PG, 128), jnp.float32),
            pltpu.VMEM_SHARED((PG, 128), jnp.float32),
        ],
    )
    def k(h2_hbm, b_hbm, sums_hbm, cnts_hbm, bidx, rows, ones_v, zbuf,
          sums_sh, cnts_sh):
        cid = lax.axis_index("c")
        sid = lax.axis_index("s")
        w = sid * NC + cid

        def fill_ones(i, _):
            for t in range(8):
                ones_v[i, pl.ds(16 * t, 16)] = jnp.full(
                    (16,), 1.0, jnp.float32)
            return 0

        lax.fori_loop(0, C, fill_ones, 0)
        _zero_vmem(zbuf, PG)

        @pl.when(sid == 0)
        def _():
            pltpu.sync_copy(zbuf, sums_sh)

        @pl.when(sid == 1)
        def _():
            pltpu.sync_copy(zbuf, cnts_sh)

        plsc.subcore_barrier()

        nw = jnp.where(w < extra, base_per_w + 1, base_per_w)
        start = w * base_per_w + jnp.minimum(w, extra)

        def body(t, _):
            c = start + t
            pltpu.sync_copy(b_hbm.at[c], bidx)
            pltpu.sync_copy(h2_hbm.at[pl.ds(c * C, C)], rows)
            pltpu.sync_copy(rows, sums_sh.at[bidx.at[0]], add=True)
            pltpu.sync_copy(ones_v, cnts_sh.at[bidx.at[0]], add=True)
            return 0

        lax.fori_loop(0, nw, body, 0)
        plsc.subcore_barrier()

        @pl.when(sid == 0)
        def _():
            pltpu.sync_copy(sums_sh, zbuf)
            pltpu.sync_copy(zbuf, sums_hbm.at[pl.ds(cid * PG, PG)])

        @pl.when(sid == 1)
        def _():
            pltpu.sync_copy(cnts_sh, zbuf)
            pltpu.sync_copy(zbuf, cnts_hbm.at[pl.ds(cid * PG, PG)])

    sums, cnts = k(h2, batch3d)
    return sums.reshape(NC, PG, 128), cnts.reshape(NC, PG, 128)


def _dinv_from(deg_ref):
    deg = deg_ref[0, :, 0:1] + deg_ref[1, :, 0:1] + 1.0
    return lax.rsqrt(deg)


def _tc1(degp, x_pad, W1, R, BLK):
    def body(deg_ref, x_ref, w_ref, o_ref):
        dinv = _dinv_from(deg_ref)
        h = jnp.dot(x_ref[...], w_ref[...], preferred_element_type=jnp.float32)
        o_ref[...] = h * dinv

    return pl.pallas_call(
        body,
        grid=(R // BLK,),
        in_specs=[
            pl.BlockSpec((2, BLK, 128), lambda i: (0, i, 0)),
            pl.BlockSpec((BLK, 128), lambda i: (i, 0)),
            pl.BlockSpec((128, 128), lambda i: (0, 0)),
        ],
        out_specs=pl.BlockSpec((BLK, 128), lambda i: (i, 0)),
        out_shape=jax.ShapeDtypeStruct((R, 128), jnp.float32),
    )(degp, x_pad, W1)


def _tc2(degp, accp, hs1, W2, b1, R, BLK):
    def body(deg_ref, acc_ref, hs_ref, w_ref, b_ref, o_ref):
        dinv = _dinv_from(deg_ref)
        t = (acc_ref[0] + acc_ref[1] + hs_ref[...]) * dinv + b_ref[...]
        h1 = jnp.maximum(t, 0.0)
        o_ref[...] = (
            jnp.dot(h1, w_ref[...], preferred_element_type=jnp.float32) * dinv
        )

    return pl.pallas_call(
        body,
        grid=(R // BLK,),
        in_specs=[
            pl.BlockSpec((2, BLK, 128), lambda i: (0, i, 0)),
            pl.BlockSpec((2, BLK, 128), lambda i: (0, i, 0)),
            pl.BlockSpec((BLK, 128), lambda i: (i, 0)),
            pl.BlockSpec((128, 128), lambda i: (0, 0)),
            pl.BlockSpec((1, 128), lambda i: (0, 0)),
        ],
        out_specs=pl.BlockSpec((BLK, 128), lambda i: (i, 0)),
        out_shape=jax.ShapeDtypeStruct((R, 128), jnp.float32),
    )(degp, accp, hs1, W2, b1)


def _tc3(degp, accp, hs2, b2, R, BLK):
    def body(deg_ref, acc_ref, hs_ref, b_ref, o_ref):
        dinv = _dinv_from(deg_ref)
        o_ref[...] = (
            (acc_ref[0] + acc_ref[1] + hs_ref[...]) * dinv + b_ref[...]
        )

    return pl.pallas_call(
        body,
        grid=(R // BLK,),
        in_specs=[
            pl.BlockSpec((2, BLK, 128), lambda i: (0, i, 0)),
            pl.BlockSpec((2, BLK, 128), lambda i: (0, i, 0)),
            pl.BlockSpec((BLK, 128), lambda i: (i, 0)),
            pl.BlockSpec((1, 128), lambda i: (0, 0)),
        ],
        out_specs=pl.BlockSpec((BLK, 128), lambda i: (i, 0)),
        out_shape=jax.ShapeDtypeStruct((R, 128), jnp.float32),
    )(degp, accp, hs2, b2)


def _tc4(sums, cnts, Wc1, bc1, Wc2, bc2, out_dim):
    def body(s_ref, c_ref, w1_ref, b1_ref, w2_ref, b2_ref, o_ref):
        s = s_ref[0, :G, :] + s_ref[1, :G, :]
        c = c_ref[0, :G, 0:1] + c_ref[1, :G, 0:1]
        gf = s / jnp.maximum(c, 1.0)
        z = jnp.dot(gf, w1_ref[...], preferred_element_type=jnp.float32)
        z = jnp.maximum(z + b1_ref[...], 0.0)
        o_ref[...] = (
            jnp.dot(z, w2_ref[...], preferred_element_type=jnp.float32)
            + b2_ref[...]
        )

    return pl.pallas_call(
        body,
        out_shape=jax.ShapeDtypeStruct((G, out_dim), jnp.float32),
    )(sums, cnts, Wc1, bc1, Wc2, bc2)


def kernel(x, edge_index, batch, W1, b1, W2, b2, Wc1, bc1, Wc2, bc2):
    N, D = x.shape
    E = edge_index.shape[1]
    out_dim = Wc2.shape[1]

    R = ((N + 1) + NS * C - 1) // (NS * C) * (NS * C)
    EC = (-(-E // (NW * C)) + 7) // 8 * 8
    E_pad = NW * C * EC

    pad_e = jnp.full((E_pad - E,), N, dtype=jnp.int32)
    srcs2d = jnp.concatenate([edge_index[0], pad_e]).reshape(NW * EC, C)
    dsts2d = jnp.concatenate([edge_index[1], pad_e]).reshape(NW * EC, C)
    x_pad = jnp.pad(x, ((0, R - N), (0, 0)))
    batch3d = jnp.concatenate(
        [batch.astype(jnp.int32), jnp.full((R - N,), G, dtype=jnp.int32)]
    ).reshape(R // C, 1, C)

    BLK = 640
    degp = _sc_degree(dsts2d, R, EC)
    hs1 = _tc1(degp, x_pad, W1, R, BLK)
    EC0, EC1 = EC, EC
    accp1 = _sc_edge_scatter(hs1, srcs2d, dsts2d, R, EC, EC0, EC1)
    hs2 = _tc2(degp, accp1, hs1, W2, b1.reshape(1, -1), R, BLK)
    accp2 = _sc_edge_scatter(hs2, srcs2d, dsts2d, R, EC, EC0, EC1)
    h2 = _tc3(degp, accp2, hs2, b2.reshape(1, -1), R, BLK)
    sums, cnts = _sc_pool(h2, batch3d, R)
    logits = _tc4(
        sums, cnts, Wc1, bc1.reshape(1, -1), Wc2, bc2.reshape(1, -1), out_dim
    )
    return logits
